# baseline (speedup 1.0000x reference)
import functools

import jax
import jax.numpy as jnp
import numpy as np
from jax import lax
from jax.experimental import pallas as pl
from jax.experimental.pallas import tpu as pltpu
from jax.experimental.pallas import tpu_sc as plsc

N = 10000
E = 160000
D = 128
DEA = 16
NH = 10000
KNN = 4

NC, NS = 2, 16
NW = NC * NS
CH = 128
EPW_RAW = E // NW
NCHUNK = 40
EPW = NCHUNK * CH
RPT = 632
NACC = NS * RPT

NHP = 10240
NP = 10240
QB = 128
GCH = 10

_SC_MESH = plsc.VectorSubcoreMesh(core_axis_name="c", subcore_axis_name="s")


def _spmm_body(x_h, src_h, dst_h, z_h, h_out, src_v, dst_v, rows_a, rows_b,
               sem_a, sem_b, acc_sh):
    c = lax.axis_index("c")
    s = lax.axis_index("s")
    w = s * NC + c
    r0 = s * RPT
    pltpu.sync_copy(z_h, acc_sh.at[pl.ds(r0, RPT)])
    plsc.subcore_barrier()
    pltpu.sync_copy(src_h.at[w], src_v)
    pltpu.sync_copy(dst_h.at[w], dst_v)

    bufs = (rows_a, rows_b)
    sems = (sem_a, sem_b)

    def gstart(j, buf, sem):
        pltpu.async_copy(x_h.at[src_v.at[j]], buf, sem)

    def gdrain(buf, sem):
        pltpu.make_async_copy(x_h.at[pl.ds(0, CH)], buf, sem).wait()

    gstart(0, rows_a, sem_a)
    gstart(1, rows_b, sem_b)

    def step(i, carry):
        for b in range(2):
            j = 2 * i + b
            gdrain(bufs[b], sems[b])

            @pl.when(j + 2 < NCHUNK)
            def _():
                gstart(j + 2, bufs[b], sems[b])

            pltpu.sync_copy(bufs[b], acc_sh.at[dst_v.at[j]], add=True)
        return carry

    lax.fori_loop(0, NCHUNK // 2, step, 0)
    plsc.subcore_barrier()
    pltpu.sync_copy(acc_sh.at[pl.ds(r0, RPT)], h_out.at[c, pl.ds(r0, RPT)])


_SPMM = pl.kernel(
    _spmm_body,
    out_type=jax.ShapeDtypeStruct((NC, NACC, D), jnp.float32),
    mesh=_SC_MESH,
    scratch_types=[
        pltpu.VMEM((NCHUNK, CH), jnp.int32),
        pltpu.VMEM((NCHUNK, CH), jnp.int32),
        pltpu.VMEM((CH, D), jnp.float32),
        pltpu.VMEM((CH, D), jnp.float32),
        pltpu.SemaphoreType.DMA,
        pltpu.SemaphoreType.DMA,
        pltpu.VMEM_SHARED((NACC, D), jnp.float32),
    ],
)


def _gather_body(y_h, idx_h, out_h, idx_v, rows_a, rows_b, sem_a, sem_b):
    c = lax.axis_index("c")
    s = lax.axis_index("s")
    w = s * NC + c
    pltpu.sync_copy(idx_h.at[w], idx_v)
    bufs = (rows_a, rows_b)
    sems = (sem_a, sem_b)

    pltpu.async_copy(y_h.at[idx_v.at[0]], rows_a, sem_a)
    pltpu.async_copy(y_h.at[idx_v.at[1]], rows_b, sem_b)

    def step(i, carry):
        for b in range(2):
            j = 2 * i + b
            pltpu.make_async_copy(y_h.at[pl.ds(0, CH)], bufs[b],
                                  sems[b]).wait()

            @pl.when(j + 2 < GCH)
            def _():
                pltpu.async_copy(y_h.at[idx_v.at[j + 2]], bufs[b], sems[b])

            pltpu.sync_copy(bufs[b], out_h.at[pl.ds((w * GCH + j) * CH, CH)])
        return carry

    lax.fori_loop(0, GCH // 2, step, 0)


_GATHER = pl.kernel(
    _gather_body,
    out_type=jax.ShapeDtypeStruct((NW * GCH * CH, D), jnp.float32),
    mesh=_SC_MESH,
    scratch_types=[
        pltpu.VMEM((GCH, CH), jnp.int32),
        pltpu.VMEM((CH, D), jnp.float32),
        pltpu.VMEM((CH, D), jnp.float32),
        pltpu.SemaphoreType.DMA,
        pltpu.SemaphoreType.DMA,
    ],
)


def _make_conv_tc(relu, skip):
    RB = 400

    def body(*refs):
        if skip:
            hp_ref, ea_ref, wx_ref, we_ref, x_ref, o_ref = refs
        else:
            hp_ref, ea_ref, wx_ref, we_ref, o_ref = refs
        h = hp_ref[0] + hp_ref[1]
        ag = ea_ref[0] + ea_ref[1]
        num = jnp.dot(h, wx_ref[...], preferred_element_type=jnp.float32)
        num = num + jnp.dot(ag, we_ref[...], preferred_element_type=jnp.float32)
        deg = ag[:, 4:5]
        out = num / jnp.maximum(deg, 1.0)
        if relu:
            out = jnp.maximum(out, 0.0)
        if skip:
            out = out + x_ref[...]
        o_ref[...] = out

    in_specs = [
        pl.BlockSpec((NC, RB, D), lambda i: (0, i, 0)),
        pl.BlockSpec((NC, RB, D), lambda i: (0, i, 0)),
        pl.BlockSpec((D, D), lambda i: (0, 0)),
        pl.BlockSpec((D, D), lambda i: (0, 0)),
    ]
    if skip:
        in_specs.append(pl.BlockSpec((RB, D), lambda i: (i, 0)))
    return pl.pallas_call(
        body,
        grid=(N // RB,),
        in_specs=in_specs,
        out_specs=pl.BlockSpec((RB, D), lambda i: (i, 0)),
        out_shape=jax.ShapeDtypeStruct((N, D), jnp.float32),
    )


_CONV_RELU = _make_conv_tc(True, False)
_CONV_SKIP = _make_conv_tc(False, True)


NG = NP // D
_BIG = float(np.float32(3e38))


def _knn_body(ph_ref, px_ref, w_ref, i_ref):
    ph = ph_ref[...]
    px = px_ref[...]
    d2 = None
    for ci in range(3):
        a = lax.broadcast_in_dim(ph[:, ci:ci + 1], (QB, NG, D), (0, 1))
        b = lax.broadcast_in_dim(px[ci:ci + 1], (QB, NG, D), (0, 1, 2))
        diff = a - b
        sq = diff * diff
        d2 = sq if d2 is None else d2 + sq
    giota = lax.broadcasted_iota(jnp.int32, (QB, NG), 1)
    liota = lax.broadcasted_iota(jnp.int32, (QB, D), 1)
    gmin = jnp.min(d2, axis=2)
    ms, ids = [], []
    picks = []
    for _ in range(KNN):
        m = jnp.min(gmin, axis=1, keepdims=True)
        g = jnp.min(jnp.where(gmin == m, giota, NG), axis=1,
                    keepdims=True)
        giota3 = lax.broadcasted_iota(jnp.int32, (QB, NG, D), 1)
        g3 = lax.broadcast_in_dim(g, (QB, NG, D), (0, 1))
        sel = jnp.min(jnp.where(giota3 == g3, d2, _BIG), axis=1)
        for (pg, plane) in picks:
            sel = jnp.where((pg == g) & (liota == plane), _BIG, sel)
        l = jnp.min(jnp.where(sel == m, liota, D), axis=1, keepdims=True)
        ms.append(m)
        ids.append(g * D + l)
        picks.append((g, l))
        gm_new = jnp.min(jnp.where(liota == l, _BIG, sel), axis=1,
                         keepdims=True)
        gmin = jnp.where(giota == g, gm_new, gmin)
    ws = [1.0 / (m + 1e-8) for m in ms]
    wtot = ws[0] + ws[1] + ws[2] + ws[3]
    wn = [wk / wtot for wk in ws]
    w_ref[...] = jnp.concatenate(
        wn + [jnp.zeros((QB, D - KNN), jnp.float32)], axis=1)
    i_ref[...] = jnp.concatenate(
        ids + [jnp.zeros((QB, D - KNN), jnp.int32)], axis=1)


_KNN = pl.pallas_call(
    _knn_body,
    grid=(NHP // QB,),
    in_specs=[
        pl.BlockSpec((QB, 8), lambda i: (i, 0)),
        pl.BlockSpec((8, NG, D), lambda i: (0, 0, 0)),
    ],
    out_specs=[
        pl.BlockSpec((QB, D), lambda i: (i, 0)),
        pl.BlockSpec((QB, D), lambda i: (i, 0)),
    ],
    out_shape=[
        jax.ShapeDtypeStruct((NHP, D), jnp.float32),
        jax.ShapeDtypeStruct((NHP, D), jnp.int32),
    ],
)


def _wsum_body(yg_ref, w_ref, o_ref):
    w = w_ref[...]
    acc = None
    for j in range(KNN):
        term = w[:, j:j + 1] * yg_ref[:, j * D:(j + 1) * D]
        acc = term if acc is None else acc + term
    o_ref[...] = acc


_WSUM = pl.pallas_call(
    _wsum_body,
    grid=(NH // 400,),
    in_specs=[
        pl.BlockSpec((400, KNN * D), lambda i: (i, 0)),
        pl.BlockSpec((400, KNN), lambda i: (i, 0)),
    ],
    out_specs=pl.BlockSpec((400, D), lambda i: (i, 0)),
    out_shape=jax.ShapeDtypeStruct((NH, D), jnp.float32),
)


def _eff_weights(W, b, alpha, din):
    al = jax.nn.softmax(alpha)
    W_eff = jnp.einsum('k,kio->io', al, W)
    dout = W.shape[2]
    we_ext = jnp.concatenate(
        [W_eff[din:din + 4], b[None, :],
         jnp.zeros((D - 5, dout), jnp.float32)], axis=0)
    return W_eff[:din], we_ext


def kernel(x, edge_index, edge_attr, pos, edge_index_high, edge_attr_high,
           pos_high, W1, b1, alpha1, W2, b2, alpha2, W3, b3, alpha3):
    x = x.astype(jnp.float32)
    src = edge_index[0].astype(jnp.int32)
    dst = edge_index[1].astype(jnp.int32)

    padi = jnp.full((NW, EPW - EPW_RAW), N, jnp.int32)
    src3 = jnp.concatenate([src.reshape(NW, EPW_RAW), padi],
                           axis=1).reshape(NW, NCHUNK, CH)
    dst3 = jnp.concatenate([dst.reshape(NW, EPW_RAW), padi],
                           axis=1).reshape(NW, NCHUNK, CH)
    ea2 = edge_attr.astype(jnp.float32).reshape(NW, EPW_RAW, 4)
    payload = jnp.concatenate(
        [ea2, jnp.ones((NW, EPW_RAW, 1), jnp.float32),
         jnp.zeros((NW, EPW_RAW, D - 5), jnp.float32)], axis=2)
    eap = jnp.concatenate(
        [payload, jnp.zeros((NW, EPW - EPW_RAW, D), jnp.float32)],
        axis=1).reshape(NW * EPW, D)
    eid3 = jnp.arange(NW * EPW, dtype=jnp.int32).reshape(NW, NCHUNK, CH)

    z128 = jnp.zeros((RPT, D), jnp.float32)
    rowpad = jnp.zeros((NACC - N, D), jnp.float32)

    w1x, w1e = _eff_weights(W1, b1, alpha1, D)
    w2x, w2e = _eff_weights(W2, b2, alpha2, D)
    w3x, w3e = _eff_weights(W3, b3, alpha3, D)

    eapart = _SPMM(eap, eid3, dst3, z128)
    ea_sl = eapart[:, :N]

    xp = jnp.concatenate([x, rowpad], axis=0)
    h1p = _SPMM(xp, src3, dst3, z128)
    e1 = _CONV_RELU(h1p[:, :N], ea_sl, w1x, w1e)

    h2p = _SPMM(jnp.concatenate([e1, rowpad], axis=0), src3, dst3, z128)
    e2 = _CONV_RELU(h2p[:, :N], ea_sl, w2x, w2e)

    h3p = _SPMM(jnp.concatenate([e2, rowpad], axis=0), src3, dst3, z128)
    y = _CONV_SKIP(h3p[:, :N], ea_sl, w3x, w3e, x)

    ph8 = jnp.concatenate(
        [pos_high.astype(jnp.float32),
         jnp.zeros((NH, 5), jnp.float32)], axis=1)
    ph8 = jnp.concatenate([ph8, jnp.zeros((NHP - NH, 8), jnp.float32)], axis=0)
    pxt = jnp.concatenate(
        [pos.astype(jnp.float32).T,
         jnp.full((3, NP - N), 1e9, jnp.float32)], axis=1)
    px8 = jnp.concatenate([pxt, jnp.zeros((5, NP), jnp.float32)],
                          axis=0).reshape(8, NG, D)
    w_pad, i_pad = _KNN(ph8, px8)
    w4 = w_pad[:NH, :KNN]
    idx4 = i_pad[:NH, :KNN]

    idxf = jnp.concatenate(
        [idx4.reshape(-1),
         jnp.zeros((NW * GCH * CH - NH * KNN,), jnp.int32)], axis=0)
    yg = _GATHER(y, idxf.reshape(NW, GCH, CH))
    yg4 = yg[:NH * KNN].reshape(NH, KNN * D)
    return _WSUM(yg4, w4)


# speedup vs baseline: 1.0412x; 1.0412x over previous
import functools

import jax
import jax.numpy as jnp
import numpy as np
from jax import lax
from jax.experimental import pallas as pl
from jax.experimental.pallas import tpu as pltpu
from jax.experimental.pallas import tpu_sc as plsc

N = 10000
E = 160000
D = 128
DEA = 16
NH = 10000
KNN = 4

NC, NS = 2, 16
NW = NC * NS
CH = 128
EPW_RAW = E // NW
NCHUNK = 40
EPW = NCHUNK * CH
RPT = 632
NACC = NS * RPT

NHP = 10240
NP = 10240
QB = 128
GCH = 10

_SC_MESH = plsc.VectorSubcoreMesh(core_axis_name="c", subcore_axis_name="s")


def _spmm_body(x_h, src_h, dst_h, z_h, h_out, src_v, dst_v, rows_a, rows_b,
               sem_a, sem_b, acc_sh):
    c = lax.axis_index("c")
    s = lax.axis_index("s")
    w = s * NC + c
    r0 = s * RPT
    pltpu.sync_copy(z_h, acc_sh.at[pl.ds(r0, RPT)])
    plsc.subcore_barrier()
    pltpu.sync_copy(src_h.at[w], src_v)
    pltpu.sync_copy(dst_h.at[w], dst_v)

    bufs = (rows_a, rows_b)
    sems = (sem_a, sem_b)

    def gstart(j, buf, sem):
        pltpu.async_copy(x_h.at[src_v.at[j]], buf, sem)

    def gdrain(buf, sem):
        pltpu.make_async_copy(x_h.at[pl.ds(0, CH)], buf, sem).wait()

    gstart(0, rows_a, sem_a)
    gstart(1, rows_b, sem_b)

    def step(i, carry):
        for b in range(2):
            j = 2 * i + b
            gdrain(bufs[b], sems[b])

            @pl.when(j + 2 < NCHUNK)
            def _():
                gstart(j + 2, bufs[b], sems[b])

            pltpu.sync_copy(bufs[b], acc_sh.at[dst_v.at[j]], add=True)
        return carry

    lax.fori_loop(0, NCHUNK // 2, step, 0)
    plsc.subcore_barrier()
    pltpu.sync_copy(acc_sh.at[pl.ds(r0, RPT)], h_out.at[c, pl.ds(r0, RPT)])


_SPMM = pl.kernel(
    _spmm_body,
    out_type=jax.ShapeDtypeStruct((NC, NACC, D), jnp.float32),
    mesh=_SC_MESH,
    scratch_types=[
        pltpu.VMEM((NCHUNK, CH), jnp.int32),
        pltpu.VMEM((NCHUNK, CH), jnp.int32),
        pltpu.VMEM((CH, D), jnp.float32),
        pltpu.VMEM((CH, D), jnp.float32),
        pltpu.SemaphoreType.DMA,
        pltpu.SemaphoreType.DMA,
        pltpu.VMEM_SHARED((NACC, D), jnp.float32),
    ],
)


def _gather_body(y_h, idx_h, out_h, idx_v, rows_a, rows_b, sem_a, sem_b):
    c = lax.axis_index("c")
    s = lax.axis_index("s")
    w = s * NC + c
    pltpu.sync_copy(idx_h.at[w], idx_v)
    bufs = (rows_a, rows_b)
    sems = (sem_a, sem_b)

    pltpu.async_copy(y_h.at[idx_v.at[0]], rows_a, sem_a)
    pltpu.async_copy(y_h.at[idx_v.at[1]], rows_b, sem_b)

    def step(i, carry):
        for b in range(2):
            j = 2 * i + b
            pltpu.make_async_copy(y_h.at[pl.ds(0, CH)], bufs[b],
                                  sems[b]).wait()

            @pl.when(j + 2 < GCH)
            def _():
                pltpu.async_copy(y_h.at[idx_v.at[j + 2]], bufs[b], sems[b])

            pltpu.sync_copy(bufs[b], out_h.at[pl.ds((w * GCH + j) * CH, CH)])
        return carry

    lax.fori_loop(0, GCH // 2, step, 0)


_GATHER = pl.kernel(
    _gather_body,
    out_type=jax.ShapeDtypeStruct((NW * GCH * CH, D), jnp.float32),
    mesh=_SC_MESH,
    scratch_types=[
        pltpu.VMEM((GCH, CH), jnp.int32),
        pltpu.VMEM((CH, D), jnp.float32),
        pltpu.VMEM((CH, D), jnp.float32),
        pltpu.SemaphoreType.DMA,
        pltpu.SemaphoreType.DMA,
    ],
)


def _make_conv_tc(relu, skip):
    RB = 400

    def body(*refs):
        if skip:
            hp_ref, ea_ref, wx_ref, we_ref, x_ref, o_ref = refs
        else:
            hp_ref, ea_ref, wx_ref, we_ref, o_ref = refs
        h = hp_ref[0] + hp_ref[1]
        ag = ea_ref[0] + ea_ref[1]
        num = jnp.dot(h, wx_ref[...], preferred_element_type=jnp.float32)
        num = num + jnp.dot(ag, we_ref[...], preferred_element_type=jnp.float32)
        deg = ag[:, 4:5]
        out = num / jnp.maximum(deg, 1.0)
        if relu:
            out = jnp.maximum(out, 0.0)
        if skip:
            out = out + x_ref[...]
        o_ref[...] = out

    in_specs = [
        pl.BlockSpec((NC, RB, D), lambda i: (0, i, 0)),
        pl.BlockSpec((NC, RB, D), lambda i: (0, i, 0)),
        pl.BlockSpec((D, D), lambda i: (0, 0)),
        pl.BlockSpec((D, D), lambda i: (0, 0)),
    ]
    if skip:
        in_specs.append(pl.BlockSpec((RB, D), lambda i: (i, 0)))
    return pl.pallas_call(
        body,
        grid=(N // RB,),
        in_specs=in_specs,
        out_specs=pl.BlockSpec((RB, D), lambda i: (i, 0)),
        out_shape=jax.ShapeDtypeStruct((N, D), jnp.float32),
    )


_CONV_RELU = _make_conv_tc(True, False)
_CONV_SKIP = _make_conv_tc(False, True)


NG = NP // D
_BIG = float(np.float32(3e38))


def _knn_body(ph_ref, px_ref, w_ref, i_ref):
    ph = ph_ref[...]
    px = px_ref[...]
    d2 = None
    for ci in range(3):
        diff = ph[:, ci:ci + 1] - px[ci:ci + 1, :]
        sq = diff * diff
        d2 = sq if d2 is None else d2 + sq
    iotaf = lax.broadcasted_iota(jnp.int32, (NP, 1), 0).astype(jnp.float32)
    ms, ids = [], []
    for _ in range(KNN):
        m = jnp.min(d2, axis=1, keepdims=True)
        eq = d2 == m
        eqf = jnp.where(eq, 1.0, 0.0)
        idxf = jnp.dot(eqf, iotaf, preferred_element_type=jnp.float32)
        ms.append(m)
        ids.append(idxf.astype(jnp.int32))
        d2 = jnp.where(eq, _BIG, d2)
    ws = [1.0 / (m + 1e-8) for m in ms]
    wtot = ws[0] + ws[1] + ws[2] + ws[3]
    wn = [wk / wtot for wk in ws]
    w_ref[...] = jnp.concatenate(
        wn + [jnp.zeros((QB, D - KNN), jnp.float32)], axis=1)
    i_ref[...] = jnp.concatenate(
        ids + [jnp.zeros((QB, D - KNN), jnp.int32)], axis=1)


_KNN = pl.pallas_call(
    _knn_body,
    grid=(NHP // QB,),
    in_specs=[
        pl.BlockSpec((QB, 8), lambda i: (i, 0)),
        pl.BlockSpec((8, NP), lambda i: (0, 0)),
    ],
    out_specs=[
        pl.BlockSpec((QB, D), lambda i: (i, 0)),
        pl.BlockSpec((QB, D), lambda i: (i, 0)),
    ],
    out_shape=[
        jax.ShapeDtypeStruct((NHP, D), jnp.float32),
        jax.ShapeDtypeStruct((NHP, D), jnp.int32),
    ],
)


def _wsum_body(yg_ref, w_ref, o_ref):
    w = w_ref[...]
    acc = None
    for j in range(KNN):
        term = w[:, j:j + 1] * yg_ref[:, j * D:(j + 1) * D]
        acc = term if acc is None else acc + term
    o_ref[...] = acc


_WSUM = pl.pallas_call(
    _wsum_body,
    grid=(NH // 400,),
    in_specs=[
        pl.BlockSpec((400, KNN * D), lambda i: (i, 0)),
        pl.BlockSpec((400, KNN), lambda i: (i, 0)),
    ],
    out_specs=pl.BlockSpec((400, D), lambda i: (i, 0)),
    out_shape=jax.ShapeDtypeStruct((NH, D), jnp.float32),
)


def _eff_weights(W, b, alpha, din):
    al = jax.nn.softmax(alpha)
    W_eff = jnp.einsum('k,kio->io', al, W)
    dout = W.shape[2]
    we_ext = jnp.concatenate(
        [W_eff[din:din + 4], b[None, :],
         jnp.zeros((D - 5, dout), jnp.float32)], axis=0)
    return W_eff[:din], we_ext


def kernel(x, edge_index, edge_attr, pos, edge_index_high, edge_attr_high,
           pos_high, W1, b1, alpha1, W2, b2, alpha2, W3, b3, alpha3):
    x = x.astype(jnp.float32)
    src = edge_index[0].astype(jnp.int32)
    dst = edge_index[1].astype(jnp.int32)

    padi = jnp.full((NW, EPW - EPW_RAW), N, jnp.int32)
    src3 = jnp.concatenate([src.reshape(NW, EPW_RAW), padi],
                           axis=1).reshape(NW, NCHUNK, CH)
    dst3 = jnp.concatenate([dst.reshape(NW, EPW_RAW), padi],
                           axis=1).reshape(NW, NCHUNK, CH)
    ea2 = edge_attr.astype(jnp.float32).reshape(NW, EPW_RAW, 4)
    payload = jnp.concatenate(
        [ea2, jnp.ones((NW, EPW_RAW, 1), jnp.float32),
         jnp.zeros((NW, EPW_RAW, D - 5), jnp.float32)], axis=2)
    eap = jnp.concatenate(
        [payload, jnp.zeros((NW, EPW - EPW_RAW, D), jnp.float32)],
        axis=1).reshape(NW * EPW, D)
    eid3 = jnp.arange(NW * EPW, dtype=jnp.int32).reshape(NW, NCHUNK, CH)

    z128 = jnp.zeros((RPT, D), jnp.float32)
    rowpad = jnp.zeros((NACC - N, D), jnp.float32)

    w1x, w1e = _eff_weights(W1, b1, alpha1, D)
    w2x, w2e = _eff_weights(W2, b2, alpha2, D)
    w3x, w3e = _eff_weights(W3, b3, alpha3, D)

    eapart = _SPMM(eap, eid3, dst3, z128)
    ea_sl = eapart[:, :N]

    xp = jnp.concatenate([x, rowpad], axis=0)
    h1p = _SPMM(xp, src3, dst3, z128)
    e1 = _CONV_RELU(h1p[:, :N], ea_sl, w1x, w1e)

    h2p = _SPMM(jnp.concatenate([e1, rowpad], axis=0), src3, dst3, z128)
    e2 = _CONV_RELU(h2p[:, :N], ea_sl, w2x, w2e)

    h3p = _SPMM(jnp.concatenate([e2, rowpad], axis=0), src3, dst3, z128)
    y = _CONV_SKIP(h3p[:, :N], ea_sl, w3x, w3e, x)

    ph8 = jnp.concatenate(
        [pos_high.astype(jnp.float32),
         jnp.zeros((NH, 5), jnp.float32)], axis=1)
    ph8 = jnp.concatenate([ph8, jnp.zeros((NHP - NH, 8), jnp.float32)], axis=0)
    pxt = jnp.concatenate(
        [pos.astype(jnp.float32).T,
         jnp.full((3, NP - N), 1e9, jnp.float32)], axis=1)
    px8 = jnp.concatenate([pxt, jnp.zeros((5, NP), jnp.float32)], axis=0)
    w_pad, i_pad = _KNN(ph8, px8)
    w4 = w_pad[:NH, :KNN]
    idx4 = jnp.clip(i_pad[:NH, :KNN], 0, N - 1)

    idxf = jnp.concatenate(
        [idx4.reshape(-1),
         jnp.zeros((NW * GCH * CH - NH * KNN,), jnp.int32)], axis=0)
    yg = _GATHER(y, idxf.reshape(NW, GCH, CH))
    yg4 = yg[:NH * KNN].reshape(NH, KNN * D)
    return _WSUM(yg4, w4)


# speedup vs baseline: 1.0682x; 1.0260x over previous
import functools

import jax
import jax.numpy as jnp
import numpy as np
from jax import lax
from jax.experimental import pallas as pl
from jax.experimental.pallas import tpu as pltpu
from jax.experimental.pallas import tpu_sc as plsc

N = 10000
E = 160000
D = 128
DEA = 16
NH = 10000
KNN = 4

NC, NS = 2, 16
NW = NC * NS
CH = 128
EPW_RAW = E // NW
NCHUNK = 40
EPW = NCHUNK * CH
RPT = 632
NACC = NS * RPT

NHP = 10240
NP = 10240
QB = 128
GCH = 10

_SC_MESH = plsc.VectorSubcoreMesh(core_axis_name="c", subcore_axis_name="s")


def _spmm_body(x_h, src_h, dst_h, z_h, h_out, src_v, dst_v, rows_a, rows_b,
               sem_a, sem_b, acc_sh):
    c = lax.axis_index("c")
    s = lax.axis_index("s")
    w = s * NC + c
    r0 = s * RPT
    pltpu.sync_copy(z_h, acc_sh.at[pl.ds(r0, RPT)])
    plsc.subcore_barrier()
    pltpu.sync_copy(src_h.at[w], src_v)
    pltpu.sync_copy(dst_h.at[w], dst_v)

    bufs = (rows_a, rows_b)
    sems = (sem_a, sem_b)

    def gstart(j, buf, sem):
        pltpu.async_copy(x_h.at[src_v.at[j]], buf, sem)

    def gdrain(buf, sem):
        pltpu.make_async_copy(x_h.at[pl.ds(0, CH)], buf, sem).wait()

    gstart(0, rows_a, sem_a)
    gstart(1, rows_b, sem_b)

    def step(i, carry):
        for b in range(2):
            j = 2 * i + b
            gdrain(bufs[b], sems[b])

            @pl.when(j + 2 < NCHUNK)
            def _():
                gstart(j + 2, bufs[b], sems[b])

            pltpu.sync_copy(bufs[b], acc_sh.at[dst_v.at[j]], add=True)
        return carry

    lax.fori_loop(0, NCHUNK // 2, step, 0)
    plsc.subcore_barrier()
    pltpu.sync_copy(acc_sh.at[pl.ds(r0, RPT)], h_out.at[c, pl.ds(r0, RPT)])


_SPMM = pl.kernel(
    _spmm_body,
    out_type=jax.ShapeDtypeStruct((NC, NACC, D), jnp.float32),
    mesh=_SC_MESH,
    scratch_types=[
        pltpu.VMEM((NCHUNK, CH), jnp.int32),
        pltpu.VMEM((NCHUNK, CH), jnp.int32),
        pltpu.VMEM((CH, D), jnp.float32),
        pltpu.VMEM((CH, D), jnp.float32),
        pltpu.SemaphoreType.DMA,
        pltpu.SemaphoreType.DMA,
        pltpu.VMEM_SHARED((NACC, D), jnp.float32),
    ],
)


def _gather_body(y_h, idx_h, out_h, idx_v, rows_a, rows_b, sem_a, sem_b):
    c = lax.axis_index("c")
    s = lax.axis_index("s")
    w = s * NC + c
    pltpu.sync_copy(idx_h.at[w], idx_v)
    bufs = (rows_a, rows_b)
    sems = (sem_a, sem_b)

    pltpu.async_copy(y_h.at[idx_v.at[0]], rows_a, sem_a)
    pltpu.async_copy(y_h.at[idx_v.at[1]], rows_b, sem_b)

    def step(i, carry):
        for b in range(2):
            j = 2 * i + b
            pltpu.make_async_copy(y_h.at[pl.ds(0, CH)], bufs[b],
                                  sems[b]).wait()

            @pl.when(j + 2 < GCH)
            def _():
                pltpu.async_copy(y_h.at[idx_v.at[j + 2]], bufs[b], sems[b])

            pltpu.sync_copy(bufs[b], out_h.at[pl.ds((w * GCH + j) * CH, CH)])
        return carry

    lax.fori_loop(0, GCH // 2, step, 0)


_GATHER = pl.kernel(
    _gather_body,
    out_type=jax.ShapeDtypeStruct((NW * GCH * CH, D), jnp.float32),
    mesh=_SC_MESH,
    scratch_types=[
        pltpu.VMEM((GCH, CH), jnp.int32),
        pltpu.VMEM((CH, D), jnp.float32),
        pltpu.VMEM((CH, D), jnp.float32),
        pltpu.SemaphoreType.DMA,
        pltpu.SemaphoreType.DMA,
    ],
)


def _make_conv_tc(relu, skip):
    RB = 400

    def body(*refs):
        if skip:
            hp_ref, ea_ref, wx_ref, we_ref, x_ref, o_ref = refs
        else:
            hp_ref, ea_ref, wx_ref, we_ref, o_ref = refs
        h = hp_ref[0] + hp_ref[1]
        ag = ea_ref[0] + ea_ref[1]
        num = jnp.dot(h, wx_ref[...], preferred_element_type=jnp.float32)
        num = num + jnp.dot(ag, we_ref[...], preferred_element_type=jnp.float32)
        deg = ag[:, 4:5]
        out = num / jnp.maximum(deg, 1.0)
        if relu:
            out = jnp.maximum(out, 0.0)
        if skip:
            out = out + x_ref[...]
        o_ref[...] = out

    in_specs = [
        pl.BlockSpec((NC, RB, D), lambda i: (0, i, 0)),
        pl.BlockSpec((NC, RB, D), lambda i: (0, i, 0)),
        pl.BlockSpec((D, D), lambda i: (0, 0)),
        pl.BlockSpec((D, D), lambda i: (0, 0)),
    ]
    if skip:
        in_specs.append(pl.BlockSpec((RB, D), lambda i: (i, 0)))
    return pl.pallas_call(
        body,
        grid=(N // RB,),
        in_specs=in_specs,
        out_specs=pl.BlockSpec((RB, D), lambda i: (i, 0)),
        out_shape=jax.ShapeDtypeStruct((N, D), jnp.float32),
    )


_CONV_RELU = _make_conv_tc(True, False)
_CONV_SKIP = _make_conv_tc(False, True)


NG = NP // D
_BIG = float(np.float32(3e38))


def _knn_body(ph_ref, px_ref, w_ref, i_ref):
    ph = ph_ref[...]
    px = px_ref[...]
    d2 = None
    for ci in range(3):
        diff = ph[:, ci:ci + 1] - px[ci:ci + 1, :]
        sq = diff * diff
        d2 = sq if d2 is None else d2 + sq
    iota = lax.broadcasted_iota(jnp.int32, (QB, NP), 1)
    ms, ids = [], []
    for _ in range(KNN):
        m = jnp.min(d2, axis=1, keepdims=True)
        sel = jnp.where(d2 == m, iota, NP)
        ij = jnp.min(sel, axis=1, keepdims=True)
        ms.append(m)
        ids.append(ij)
        d2 = jnp.where(iota == ij, _BIG, d2)
    ws = [1.0 / (m + 1e-8) for m in ms]
    wtot = ws[0] + ws[1] + ws[2] + ws[3]
    wn = [wk / wtot for wk in ws]
    w_ref[...] = jnp.concatenate(
        wn + [jnp.zeros((QB, D - KNN), jnp.float32)], axis=1)
    i_ref[...] = jnp.concatenate(
        ids + [jnp.zeros((QB, D - KNN), jnp.int32)], axis=1)


_KNN = pl.pallas_call(
    _knn_body,
    grid=(NHP // QB,),
    in_specs=[
        pl.BlockSpec((QB, 8), lambda i: (i, 0)),
        pl.BlockSpec((8, NP), lambda i: (0, 0)),
    ],
    out_specs=[
        pl.BlockSpec((QB, D), lambda i: (i, 0)),
        pl.BlockSpec((QB, D), lambda i: (i, 0)),
    ],
    out_shape=[
        jax.ShapeDtypeStruct((NHP, D), jnp.float32),
        jax.ShapeDtypeStruct((NHP, D), jnp.int32),
    ],
)


def _wsum_body(yg_ref, w_ref, o_ref):
    w = w_ref[...]
    acc = None
    for j in range(KNN):
        term = w[:, j:j + 1] * yg_ref[:, j * D:(j + 1) * D]
        acc = term if acc is None else acc + term
    o_ref[...] = acc


_WSUM = pl.pallas_call(
    _wsum_body,
    grid=(NH // 400,),
    in_specs=[
        pl.BlockSpec((400, KNN * D), lambda i: (i, 0)),
        pl.BlockSpec((400, KNN), lambda i: (i, 0)),
    ],
    out_specs=pl.BlockSpec((400, D), lambda i: (i, 0)),
    out_shape=jax.ShapeDtypeStruct((NH, D), jnp.float32),
)


def _eff_weights(W, b, alpha, din):
    al = jax.nn.softmax(alpha)
    W_eff = jnp.einsum('k,kio->io', al, W)
    dout = W.shape[2]
    we_ext = jnp.concatenate(
        [W_eff[din:din + 4], b[None, :],
         jnp.zeros((D - 5, dout), jnp.float32)], axis=0)
    return W_eff[:din], we_ext


def kernel(x, edge_index, edge_attr, pos, edge_index_high, edge_attr_high,
           pos_high, W1, b1, alpha1, W2, b2, alpha2, W3, b3, alpha3):
    x = x.astype(jnp.float32)
    src = edge_index[0].astype(jnp.int32)
    dst = edge_index[1].astype(jnp.int32)

    padi = jnp.full((NW, EPW - EPW_RAW), N, jnp.int32)
    src3 = jnp.concatenate([src.reshape(NW, EPW_RAW), padi],
                           axis=1).reshape(NW, NCHUNK, CH)
    dst3 = jnp.concatenate([dst.reshape(NW, EPW_RAW), padi],
                           axis=1).reshape(NW, NCHUNK, CH)
    ea2 = edge_attr.astype(jnp.float32).reshape(NW, EPW_RAW, 4)
    payload = jnp.concatenate(
        [ea2, jnp.ones((NW, EPW_RAW, 1), jnp.float32),
         jnp.zeros((NW, EPW_RAW, D - 5), jnp.float32)], axis=2)
    eap = jnp.concatenate(
        [payload, jnp.zeros((NW, EPW - EPW_RAW, D), jnp.float32)],
        axis=1).reshape(NW * EPW, D)
    eid3 = jnp.arange(NW * EPW, dtype=jnp.int32).reshape(NW, NCHUNK, CH)

    z128 = jnp.zeros((RPT, D), jnp.float32)
    rowpad = jnp.zeros((NACC - N, D), jnp.float32)

    w1x, w1e = _eff_weights(W1, b1, alpha1, D)
    w2x, w2e = _eff_weights(W2, b2, alpha2, D)
    w3x, w3e = _eff_weights(W3, b3, alpha3, D)

    ph8 = jnp.concatenate(
        [pos_high.astype(jnp.float32),
         jnp.zeros((NH, 5), jnp.float32)], axis=1)
    ph8 = jnp.concatenate([ph8, jnp.zeros((NHP - NH, 8), jnp.float32)], axis=0)
    pxt = jnp.concatenate(
        [pos.astype(jnp.float32).T,
         jnp.full((3, NP - N), 1e9, jnp.float32)], axis=1)
    px8 = jnp.concatenate([pxt, jnp.zeros((5, NP), jnp.float32)], axis=0)
    w_pad, i_pad = _KNN(ph8, px8)
    w4 = w_pad[:NH, :KNN]
    idx4 = jnp.clip(i_pad[:NH, :KNN], 0, N - 1)
    idxf = jnp.concatenate(
        [idx4.reshape(-1),
         jnp.zeros((NW * GCH * CH - NH * KNN,), jnp.int32)], axis=0)

    eapart = _SPMM(eap, eid3, dst3, z128)
    ea_sl = eapart[:, :N]

    xp = jnp.concatenate([x, rowpad], axis=0)
    h1p = _SPMM(xp, src3, dst3, z128)
    e1 = _CONV_RELU(h1p[:, :N], ea_sl, w1x, w1e)

    h2p = _SPMM(jnp.concatenate([e1, rowpad], axis=0), src3, dst3, z128)
    e2 = _CONV_RELU(h2p[:, :N], ea_sl, w2x, w2e)

    h3p = _SPMM(jnp.concatenate([e2, rowpad], axis=0), src3, dst3, z128)
    y = _CONV_SKIP(h3p[:, :N], ea_sl, w3x, w3e, x)

    yg = _GATHER(y, idxf.reshape(NW, GCH, CH))
    yg4 = yg[:NH * KNN].reshape(NH, KNN * D)
    return _WSUM(yg4, w4)
